# Initial kernel scaffold; baseline (speedup 1.0000x reference)
import jax
import jax.numpy as jnp
from jax.experimental import pallas as pl


def kernel(edge_attr, emb0, emb1, emb2):
    raise NotImplementedError("write your pallas kernel here")


# speedup vs baseline: 8.9219x; 8.9219x over previous
import functools

import jax
import jax.numpy as jnp
from jax import lax
from jax.experimental import pallas as pl
from jax.experimental.pallas import tpu as pltpu
from jax.experimental.pallas import tpu_sc as plsc

E = 320000
D = 128
NCAT = 14
T_ROWS = NCAT * NCAT * NCAT

NC = 2
NS = 16
NW = NC * NS
R_PER_W = E // NW
GROUP = 400
N_GROUPS = R_PER_W // GROUP
DMA_B = 80
N_DMA = GROUP // DMA_B
JSTEPS = GROUP // 16


def _build_table(e0, e1, e2):
    def body(e0_ref, e1_ref, e2_ref, t_ref):
        t_ref[...] = (
            e0_ref[...][:, None, None, :]
            + e1_ref[...][None, :, None, :]
        ) + e2_ref[...][None, None, :, :]

    t4 = pl.pallas_call(
        body,
        out_shape=jax.ShapeDtypeStruct((NCAT, NCAT, NCAT, D), jnp.float32),
    )(e0, e1, e2)
    return t4.reshape(T_ROWS, D)


_mesh = plsc.VectorSubcoreMesh(core_axis_name="c", subcore_axis_name="s")


@functools.partial(
    pl.kernel,
    mesh=_mesh,
    out_type=jax.ShapeDtypeStruct((E, D), jnp.float32),
    scratch_types=[
        pltpu.VMEM((GROUP,), jnp.int32),
        pltpu.VMEM((GROUP,), jnp.int32),
        pltpu.VMEM((GROUP,), jnp.int32),
        pltpu.VMEM((N_DMA, DMA_B), jnp.int32),
        pltpu.VMEM((GROUP, D), jnp.float32),
        pltpu.SemaphoreType.DMA,
    ],
)
def _sc_lookup(ea0_hbm, ea1_hbm, ea2_hbm, t_hbm, out_hbm,
               e0_v, e1_v, e2_v, c_v, rows_v, sem):
    wid = lax.axis_index("s") * NC + lax.axis_index("c")
    base = wid * R_PER_W

    def group(g, carry):
        gbase = base + g * GROUP
        pltpu.sync_copy(ea0_hbm.at[pl.ds(gbase, GROUP)], e0_v)
        pltpu.sync_copy(ea1_hbm.at[pl.ds(gbase, GROUP)], e1_v)
        pltpu.sync_copy(ea2_hbm.at[pl.ds(gbase, GROUP)], e2_v)
        for j in range(JSTEPS):
            a0 = e0_v[pl.ds(j * 16, 16)]
            a1 = e1_v[pl.ds(j * 16, 16)]
            a2 = e2_v[pl.ds(j * 16, 16)]
            c = (a0 * NCAT + a1) * NCAT + a2
            c_v[j // 5, pl.ds((j % 5) * 16, 16)] = c
        copies = [
            pltpu.async_copy(
                t_hbm.at[c_v.at[b]],
                rows_v.at[pl.ds(b * DMA_B, DMA_B)],
                sem,
            )
            for b in range(N_DMA)
        ]
        for cp in copies:
            cp.wait()
        pltpu.sync_copy(rows_v, out_hbm.at[pl.ds(gbase, GROUP)])
        return carry

    lax.fori_loop(0, N_GROUPS, group, None)


def kernel(edge_attr, emb0, emb1, emb2):
    ea = edge_attr.astype(jnp.int32)
    ea0 = ea[:, 0]
    ea1 = ea[:, 1]
    ea2 = ea[:, 2]
    t = _build_table(emb0, emb1, emb2)
    return _sc_lookup(ea0, ea1, ea2, t)


# speedup vs baseline: 10.8909x; 1.2207x over previous
import functools

import jax
import jax.numpy as jnp
from jax import lax
from jax.experimental import pallas as pl
from jax.experimental.pallas import tpu as pltpu
from jax.experimental.pallas import tpu_sc as plsc

E = 320000
D = 128
NCAT = 14
T_ROWS = NCAT * NCAT * NCAT

NC = 2
NS = 16
NW = NC * NS
R_PER_W = E // NW
GROUP = 400
N_GROUPS = R_PER_W // GROUP
DMA_B = 80
N_DMA = GROUP // DMA_B
JSTEPS = GROUP // 16


def _build_table(e0, e1, e2):
    def body(e0_ref, e1_ref, e2_ref, t_ref):
        t_ref[...] = (
            e0_ref[...][:, None, None, :]
            + e1_ref[...][None, :, None, :]
        ) + e2_ref[...][None, None, :, :]

    t4 = pl.pallas_call(
        body,
        out_shape=jax.ShapeDtypeStruct((NCAT, NCAT, NCAT, D), jnp.float32),
    )(e0, e1, e2)
    return t4.reshape(T_ROWS, D)


_mesh = plsc.VectorSubcoreMesh(core_axis_name="c", subcore_axis_name="s")


@functools.partial(
    pl.kernel,
    mesh=_mesh,
    out_type=jax.ShapeDtypeStruct((E, D), jnp.float32),
    scratch_types=[
        pltpu.VMEM((GROUP,), jnp.int32),
        pltpu.VMEM((GROUP,), jnp.int32),
        pltpu.VMEM((GROUP,), jnp.int32),
        pltpu.VMEM((GROUP,), jnp.int32),
        pltpu.VMEM((GROUP,), jnp.int32),
        pltpu.VMEM((GROUP,), jnp.int32),
        pltpu.VMEM((N_DMA, DMA_B), jnp.int32),
        pltpu.VMEM((N_DMA, DMA_B), jnp.int32),
        pltpu.VMEM((GROUP, D), jnp.float32),
        pltpu.VMEM((GROUP, D), jnp.float32),
        pltpu.SemaphoreType.DMA,
        pltpu.SemaphoreType.DMA,
        pltpu.SemaphoreType.DMA,
        pltpu.SemaphoreType.DMA,
        pltpu.SemaphoreType.DMA,
        pltpu.SemaphoreType.DMA,
    ],
)
def _sc_lookup(ea0_hbm, ea1_hbm, ea2_hbm, t_hbm, out_hbm,
               e00, e01, e02, e10, e11, e12, c0, c1, r0, r1,
               l0, l1, g0, g1, s0, s1):
    wid = lax.axis_index("s") * NC + lax.axis_index("c")
    base = wid * R_PER_W
    ebufs = ((e00, e01, e02), (e10, e11, e12))
    cbufs = (c0, c1)
    rbufs = (r0, r1)
    lsems = (l0, l1)
    gsems = (g0, g1)
    ssems = (s0, s1)

    def fire_cols(g):
        p = g % 2
        gbase = base + g * GROUP
        return [
            pltpu.async_copy(eah.at[pl.ds(gbase, GROUP)], ebufs[p][k],
                             lsems[p])
            for k, eah in enumerate((ea0_hbm, ea1_hbm, ea2_hbm))
        ]

    col_copies = {0: fire_cols(0)}
    store_copies = {}

    for g in range(N_GROUPS):
        p = g % 2
        gbase = base + g * GROUP
        for cp in col_copies.pop(g):
            cp.wait()
        for j in range(JSTEPS):
            a0 = ebufs[p][0][pl.ds(j * 16, 16)]
            a1 = ebufs[p][1][pl.ds(j * 16, 16)]
            a2 = ebufs[p][2][pl.ds(j * 16, 16)]
            c = (a0 * NCAT + a1) * NCAT + a2
            cbufs[p][j // 5, pl.ds((j % 5) * 16, 16)] = c
        if g >= 2:
            store_copies.pop(g - 2).wait()
        gathers = [
            pltpu.async_copy(
                t_hbm.at[cbufs[p].at[b]],
                rbufs[p].at[pl.ds(b * DMA_B, DMA_B)],
                gsems[p],
            )
            for b in range(N_DMA)
        ]
        if g + 1 < N_GROUPS:
            col_copies[g + 1] = fire_cols(g + 1)
        for cp in gathers:
            cp.wait()
        store_copies[g] = pltpu.async_copy(
            rbufs[p], out_hbm.at[pl.ds(gbase, GROUP)], ssems[p])

    for g in (N_GROUPS - 2, N_GROUPS - 1):
        store_copies.pop(g).wait()


def kernel(edge_attr, emb0, emb1, emb2):
    ea = edge_attr.astype(jnp.int32)
    ea0 = ea[:, 0]
    ea1 = ea[:, 1]
    ea2 = ea[:, 2]
    t = _build_table(emb0, emb1, emb2)
    return _sc_lookup(ea0, ea1, ea2, t)
